# baseline (speedup 1.0000x reference)
import functools

import jax
import jax.numpy as jnp
from jax import lax
from jax.experimental import pallas as pl
from jax.experimental.pallas import tpu as pltpu
from jax.experimental.pallas import tpu_sc as plsc

_NC = 2
_NS = 16
_NW = _NC * _NS
_L = 16
_BIG = 1 << 20


@functools.lru_cache(maxsize=None)
def _build(B, S, V, D, N):
    T = B * S
    CHUNK = T // _NW
    CPR = S // CHUNK
    SUB = 128
    NSUB = CHUNK // SUB
    NG = CHUNK // _L
    NCOL = D // _L
    assert T % _NW == 0 and CHUNK % SUB == 0 and D % _L == 0
    assert S % CHUNK == 0
    assert N <= _L

    mesh = plsc.VectorSubcoreMesh(core_axis_name="c", subcore_axis_name="s")

    @functools.partial(
        pl.kernel,
        mesh=mesh,
        compiler_params=pltpu.CompilerParams(needs_layout_passes=False),
        out_type=jax.ShapeDtypeStruct((B, S, D), jnp.float32),
        scratch_types=[
            pltpu.VMEM((B, S), jnp.int32),
            pltpu.VMEM((N * D,), jnp.float32),
            pltpu.VMEM((CHUNK, D), jnp.float32),
            pltpu.SemaphoreType.DMA((NSUB,)),
            pltpu.SemaphoreType.DMA((NSUB,)),
        ],
    )
    def _soft_embed(tok_hbm, wte_hbm, learned_hbm, out_hbm,
                    tok_v, learned_v, rows_v, gsem, wsem):
        wid = lax.axis_index("s") * _NC + lax.axis_index("c")
        b = wid // CPR
        col = (wid % CPR) * CHUNK

        pltpu.sync_copy(tok_hbm, tok_v)

        gcps = [
            pltpu.async_copy(
                wte_hbm.at[tok_v.at[b, pl.ds(col + q * SUB, SUB)]],
                rows_v.at[pl.ds(q * SUB, SUB)],
                gsem.at[q],
            )
            for q in range(NSUB)
        ]

        lane = lax.iota(jnp.int32, _L)
        loc_v = jnp.full((_L,), _BIG, jnp.int32)
        for q in range(NG):
            t = tok_v[b, pl.ds(col + q * _L, _L)]
            m = t == 0
            c = plsc.all_reduce_population_count(m)
            f = plsc.all_reduce_ffs(m)
            loc_v = jnp.minimum(loc_v, jnp.where(c > 0, f + q * _L, _BIG))
        w2 = wid + 1
        b2 = jnp.minimum(w2 // CPR, B - 1)
        c2 = (w2 % CPR) * CHUNK
        t = tok_v[b2, pl.ds(c2, _L)]
        m = jnp.logical_and(jnp.logical_and(t == 0, lane < N), wid < _NW - 1)
        c = plsc.all_reduce_population_count(m)
        f = plsc.all_reduce_ffs(m)
        loc_v = jnp.minimum(loc_v, jnp.where(c > 0, f + CHUNK, _BIG))

        loc = jnp.min(loc_v)

        @pl.when(loc - N < CHUNK)
        def _():
            pltpu.sync_copy(learned_hbm, learned_v)

        wcps = []
        for q in range(NSUB):
            gcps[q].wait()
            lo, hi = q * SUB, (q + 1) * SUB

            @pl.when(jnp.logical_and(loc - N < hi, loc > lo))
            def _():
                for o in range(N):
                    pv = loc_v - (N - o)
                    mk = jnp.logical_and(pv >= lo, pv < hi)
                    for cg in range(NCOL):
                        cols = lane + cg * _L
                        src = learned_v[pl.ds(o * D + cg * _L, _L)]
                        plsc.store_scatter(rows_v, [pv, cols], src, mask=mk)

            wcps.append(
                pltpu.async_copy(
                    rows_v.at[pl.ds(lo, SUB)],
                    out_hbm.at[b, pl.ds(col + lo, SUB)],
                    wsem.at[q],
                )
            )
        for cp in wcps:
            cp.wait()

    return _soft_embed


def kernel(tokens, wte_weight, learned_embedding):
    B, S = tokens.shape
    V, D = wte_weight.shape
    N = learned_embedding.shape[0]
    k = _build(B, S, V, D, N)
    return k(tokens.astype(jnp.int32),
             wte_weight.astype(jnp.float32),
             learned_embedding.astype(jnp.float32).reshape(-1))


# speedup vs baseline: 1.0872x; 1.0872x over previous
import functools

import jax
import jax.numpy as jnp
from jax import lax
from jax.experimental import pallas as pl
from jax.experimental.pallas import tpu as pltpu
from jax.experimental.pallas import tpu_sc as plsc

_NC = 2
_NS = 16
_NW = _NC * _NS
_L = 16
_BIG = 1 << 20


@functools.lru_cache(maxsize=None)
def _build(B, S, V, D, N):
    T = B * S
    CHUNK = T // _NW
    CPR = S // CHUNK
    SUB = 128
    NSUB = CHUNK // SUB
    NG = CHUNK // _L
    NCOL = D // _L
    assert T % _NW == 0 and CHUNK % SUB == 0 and D % _L == 0
    assert S % CHUNK == 0
    assert N <= _L

    mesh = plsc.VectorSubcoreMesh(core_axis_name="c", subcore_axis_name="s")

    @functools.partial(
        pl.kernel,
        mesh=mesh,
        compiler_params=pltpu.CompilerParams(needs_layout_passes=False),
        out_type=jax.ShapeDtypeStruct((B, S, D), jnp.float32),
        scratch_types=[
            pltpu.VMEM((B, CHUNK + 128), jnp.int32),
            pltpu.VMEM((N * D,), jnp.float32),
            pltpu.VMEM((CHUNK, D), jnp.float32),
            pltpu.SemaphoreType.DMA((NSUB,)),
            pltpu.SemaphoreType.DMA((NSUB,)),
        ],
    )
    def _soft_embed(tok_hbm, wte_hbm, learned_hbm, out_hbm,
                    tok_v, learned_v, rows_v, gsem, wsem):
        wid = lax.axis_index("s") * _NC + lax.axis_index("c")
        b = wid // CPR
        col = (wid % CPR) * CHUNK

        row_last = col == S - CHUNK

        @pl.when(jnp.logical_not(row_last))
        def _():
            pltpu.sync_copy(tok_hbm.at[pl.ds(0, B), pl.ds(col, CHUNK + 128)],
                            tok_v)

        @pl.when(row_last)
        def _():
            pltpu.sync_copy(tok_hbm.at[pl.ds(0, B), pl.ds(col, CHUNK)],
                            tok_v.at[pl.ds(0, B), pl.ds(0, CHUNK)])
            pltpu.sync_copy(tok_hbm.at[pl.ds(0, B), pl.ds(0, 128)],
                            tok_v.at[pl.ds(0, B), pl.ds(CHUNK, 128)])

        gcps = [
            pltpu.async_copy(
                wte_hbm.at[tok_v.at[b, pl.ds(q * SUB, SUB)]],
                rows_v.at[pl.ds(q * SUB, SUB)],
                gsem.at[q],
            )
            for q in range(NSUB)
        ]

        lane = lax.iota(jnp.int32, _L)
        loc_v = jnp.full((_L,), _BIG, jnp.int32)
        for q in range(NG):
            t = tok_v[b, pl.ds(q * _L, _L)]
            m = t == 0
            c = plsc.all_reduce_population_count(m)
            f = plsc.all_reduce_ffs(m)
            loc_v = jnp.minimum(loc_v, jnp.where(c > 0, f + q * _L, _BIG))
        b2 = jnp.where(row_last, jnp.minimum(b + 1, B - 1), b)
        t = tok_v[b2, pl.ds(CHUNK, _L)]
        m = jnp.logical_and(jnp.logical_and(t == 0, lane < N), wid < _NW - 1)
        c = plsc.all_reduce_population_count(m)
        f = plsc.all_reduce_ffs(m)
        loc_v = jnp.minimum(loc_v, jnp.where(c > 0, f + CHUNK, _BIG))

        loc = jnp.min(loc_v)

        @pl.when(loc - N < CHUNK)
        def _():
            pltpu.sync_copy(learned_hbm, learned_v)

        wcps = []
        for q in range(NSUB):
            gcps[q].wait()
            lo, hi = q * SUB, (q + 1) * SUB

            @pl.when(jnp.logical_and(loc - N < hi, loc > lo))
            def _():
                for o in range(N):
                    pv = loc_v - (N - o)
                    mk = jnp.logical_and(pv >= lo, pv < hi)
                    for cg in range(NCOL):
                        cols = lane + cg * _L
                        src = learned_v[pl.ds(o * D + cg * _L, _L)]
                        plsc.store_scatter(rows_v, [pv, cols], src, mask=mk)

            wcps.append(
                pltpu.async_copy(
                    rows_v.at[pl.ds(lo, SUB)],
                    out_hbm.at[b, pl.ds(col + lo, SUB)],
                    wsem.at[q],
                )
            )
        for cp in wcps:
            cp.wait()

    return _soft_embed


def kernel(tokens, wte_weight, learned_embedding):
    B, S = tokens.shape
    V, D = wte_weight.shape
    N = learned_embedding.shape[0]
    k = _build(B, S, V, D, N)
    return k(tokens.astype(jnp.int32),
             wte_weight.astype(jnp.float32),
             learned_embedding.astype(jnp.float32).reshape(-1))


# speedup vs baseline: 1.0947x; 1.0068x over previous
import functools

import jax
import jax.numpy as jnp
from jax import lax
from jax.experimental import pallas as pl
from jax.experimental.pallas import tpu as pltpu
from jax.experimental.pallas import tpu_sc as plsc

_NC = 2
_NS = 16
_NW = _NC * _NS
_L = 16
_BIG = 1 << 20


@functools.lru_cache(maxsize=None)
def _build(B, S, V, D, N):
    T = B * S
    CHUNK = T // _NW
    CPR = S // CHUNK
    SUB = 128
    NSUB = CHUNK // SUB
    NG = CHUNK // _L
    NCOL = D // _L
    assert T % _NW == 0 and CHUNK % SUB == 0 and D % _L == 0
    assert S % CHUNK == 0
    assert N <= _L

    mesh = plsc.VectorSubcoreMesh(core_axis_name="c", subcore_axis_name="s")

    @functools.partial(
        pl.kernel,
        mesh=mesh,
        compiler_params=pltpu.CompilerParams(needs_layout_passes=False),
        out_type=jax.ShapeDtypeStruct((B, S, D), jnp.float32),
        scratch_types=[
            pltpu.VMEM((B, CHUNK + 128), jnp.int32),
            pltpu.VMEM((N * D,), jnp.float32),
            pltpu.VMEM((CHUNK, D), jnp.float32),
            pltpu.SemaphoreType.DMA((NSUB,)),
            pltpu.SemaphoreType.DMA((NSUB,)),
            pltpu.SemaphoreType.DMA((2,)),
        ],
    )
    def _soft_embed(tok_hbm, wte_hbm, learned_hbm, out_hbm,
                    tok_v, learned_v, rows_v, gsem, wsem, tsem):
        wid = lax.axis_index("s") * _NC + lax.axis_index("c")
        b = wid // CPR
        col = pl.multiple_of((wid % CPR) * CHUNK, 128)

        row_last = col == S - CHUNK
        col2 = pl.multiple_of(jnp.where(row_last, 0, col + CHUNK), 128)
        cp_main = pltpu.async_copy(
            tok_hbm.at[pl.ds(0, B), pl.ds(col, CHUNK)],
            tok_v.at[pl.ds(0, B), pl.ds(0, CHUNK)], tsem.at[0])
        cp_look = pltpu.async_copy(
            tok_hbm.at[pl.ds(0, B), pl.ds(col2, 128)],
            tok_v.at[pl.ds(0, B), pl.ds(CHUNK, 128)], tsem.at[1])
        cp_main.wait()

        gcps = [
            pltpu.async_copy(
                wte_hbm.at[tok_v.at[b, pl.ds(q * SUB, SUB)]],
                rows_v.at[pl.ds(q * SUB, SUB)],
                gsem.at[q],
            )
            for q in range(NSUB)
        ]

        lane = lax.iota(jnp.int32, _L)
        loc_v = jnp.full((_L,), _BIG, jnp.int32)
        for q in range(NG):
            t = tok_v[b, pl.ds(q * _L, _L)]
            m = t == 0
            c = plsc.all_reduce_population_count(m)
            f = plsc.all_reduce_ffs(m)
            loc_v = jnp.minimum(loc_v, jnp.where(c > 0, f + q * _L, _BIG))
        cp_look.wait()
        b2 = jnp.where(row_last, jnp.minimum(b + 1, B - 1), b)
        t = tok_v[b2, pl.ds(CHUNK, _L)]
        m = jnp.logical_and(jnp.logical_and(t == 0, lane < N), wid < _NW - 1)
        c = plsc.all_reduce_population_count(m)
        f = plsc.all_reduce_ffs(m)
        loc_v = jnp.minimum(loc_v, jnp.where(c > 0, f + CHUNK, _BIG))

        loc = jnp.min(loc_v)

        @pl.when(loc - N < CHUNK)
        def _():
            pltpu.sync_copy(learned_hbm, learned_v)

        wcps = []
        for q in range(NSUB):
            gcps[q].wait()
            lo, hi = q * SUB, (q + 1) * SUB

            @pl.when(jnp.logical_and(loc - N < hi, loc > lo))
            def _():
                for o in range(N):
                    pv = loc_v - (N - o)
                    mk = jnp.logical_and(pv >= lo, pv < hi)
                    for cg in range(NCOL):
                        cols = lane + cg * _L
                        src = learned_v[pl.ds(o * D + cg * _L, _L)]
                        plsc.store_scatter(rows_v, [pv, cols], src, mask=mk)

            wcps.append(
                pltpu.async_copy(
                    rows_v.at[pl.ds(lo, SUB)],
                    out_hbm.at[b, pl.ds(col + lo, SUB)],
                    wsem.at[q],
                )
            )
        for cp in wcps:
            cp.wait()

    return _soft_embed


def kernel(tokens, wte_weight, learned_embedding):
    B, S = tokens.shape
    V, D = wte_weight.shape
    N = learned_embedding.shape[0]
    k = _build(B, S, V, D, N)
    return k(tokens.astype(jnp.int32),
             wte_weight.astype(jnp.float32),
             learned_embedding.astype(jnp.float32).reshape(-1))


# speedup vs baseline: 1.1203x; 1.0234x over previous
import functools

import jax
import jax.numpy as jnp
from jax import lax
from jax.experimental import pallas as pl
from jax.experimental.pallas import tpu as pltpu
from jax.experimental.pallas import tpu_sc as plsc

_NC = 2
_NS = 16
_NW = _NC * _NS
_L = 16
_BIG = 1 << 20


@functools.lru_cache(maxsize=None)
def _build(B, S, V, D, N):
    T = B * S
    CHUNK = T // _NW
    CPR = S // CHUNK
    SUB = 128
    NSUB = CHUNK // SUB
    NG = CHUNK // _L
    NCOL = D // _L
    assert T % _NW == 0 and CHUNK % SUB == 0 and D % _L == 0
    assert S % CHUNK == 0
    assert N <= _L

    mesh = plsc.VectorSubcoreMesh(core_axis_name="c", subcore_axis_name="s")

    @functools.partial(
        pl.kernel,
        mesh=mesh,
        compiler_params=pltpu.CompilerParams(needs_layout_passes=False),
        out_type=jax.ShapeDtypeStruct((B, S, D), jnp.float32),
        scratch_types=[
            pltpu.VMEM((B, CHUNK + 128), jnp.int32),
            pltpu.VMEM((N * D,), jnp.float32),
            pltpu.VMEM((CHUNK, D), jnp.float32),
            pltpu.SemaphoreType.DMA((NSUB,)),
            pltpu.SemaphoreType.DMA((NSUB,)),
            pltpu.SemaphoreType.DMA((2,)),
        ],
    )
    def _soft_embed(tok_hbm, wte_hbm, learned_hbm, out_hbm,
                    tok_v, learned_v, rows_v, gsem, wsem, tsem):
        wid = lax.axis_index("s") * _NC + lax.axis_index("c")
        b = wid // CPR
        col = pl.multiple_of((wid % CPR) * CHUNK, 128)

        row_last = col == S - CHUNK
        col2 = pl.multiple_of(jnp.where(row_last, 0, col + CHUNK), 128)
        cp_main = pltpu.async_copy(
            tok_hbm.at[pl.ds(0, B), pl.ds(col, CHUNK)],
            tok_v.at[pl.ds(0, B), pl.ds(0, CHUNK)], tsem.at[0])
        cp_look = pltpu.async_copy(
            tok_hbm.at[pl.ds(0, B), pl.ds(col2, 128)],
            tok_v.at[pl.ds(0, B), pl.ds(CHUNK, 128)], tsem.at[1])
        cp_main.wait()

        gcps = [
            pltpu.async_copy(
                wte_hbm.at[tok_v.at[b, pl.ds(q * SUB, SUB)]],
                rows_v.at[pl.ds(q * SUB, SUB)],
                gsem.at[q],
            )
            for q in range(NSUB)
        ]

        lane = lax.iota(jnp.int32, _L)

        def _scan_group(q, acc):
            t = tok_v[b, pl.ds(q * _L, _L)]
            m = t == 0
            c = plsc.all_reduce_population_count(m)
            f = plsc.all_reduce_ffs(m)
            return jnp.minimum(acc, jnp.where(c > 0, f + q * _L, _BIG))

        loc_v = lax.fori_loop(0, NG, _scan_group,
                              jnp.full((_L,), _BIG, jnp.int32))
        cp_look.wait()
        b2 = jnp.where(row_last, jnp.minimum(b + 1, B - 1), b)
        t = tok_v[b2, pl.ds(CHUNK, _L)]
        m = jnp.logical_and(jnp.logical_and(t == 0, lane < N), wid < _NW - 1)
        c = plsc.all_reduce_population_count(m)
        f = plsc.all_reduce_ffs(m)
        loc_v = jnp.minimum(loc_v, jnp.where(c > 0, f + CHUNK, _BIG))

        loc = jnp.min(loc_v)

        @pl.when(loc - N < CHUNK)
        def _():
            pltpu.sync_copy(learned_hbm, learned_v)

        wcps = []
        for q in range(NSUB):
            gcps[q].wait()
            lo, hi = q * SUB, (q + 1) * SUB

            @pl.when(jnp.logical_and(loc - N < hi, loc > lo))
            def _():
                def _patch_row(o, carry):
                    pv = loc_v - (N - o)
                    mk = jnp.logical_and(pv >= lo, pv < hi)
                    for cg in range(NCOL):
                        cols = lane + cg * _L
                        src = learned_v[pl.ds(o * D + cg * _L, _L)]
                        plsc.store_scatter(rows_v, [pv, cols], src, mask=mk)
                    return carry

                lax.fori_loop(0, N, _patch_row, 0)

            wcps.append(
                pltpu.async_copy(
                    rows_v.at[pl.ds(lo, SUB)],
                    out_hbm.at[b, pl.ds(col + lo, SUB)],
                    wsem.at[q],
                )
            )
        for cp in wcps:
            cp.wait()

    return _soft_embed


def kernel(tokens, wte_weight, learned_embedding):
    B, S = tokens.shape
    V, D = wte_weight.shape
    N = learned_embedding.shape[0]
    k = _build(B, S, V, D, N)
    return k(tokens.astype(jnp.int32),
             wte_weight.astype(jnp.float32),
             learned_embedding.astype(jnp.float32).reshape(-1))
